# baseline (speedup 1.0000x reference)
import functools

import jax
import jax.numpy as jnp
from jax import lax
from jax.experimental import pallas as pl
from jax.experimental.pallas import tpu as pltpu
from jax.experimental.pallas import tpu_sc as plsc

EPS = 1e-7

N_ANCHORS = 36864
NC, NS, L = 2, 16, 16
NW = NC * NS
APW = N_ANCHORS // NW
CPW = APW * 4

LN2 = 0.6931471805599453
_C1 = 1.4425170540492982
_C2 = -0.7178986854566598
_C3 = 0.4568956949250845
_C4 = -0.2773683961533664
_C5 = 0.12191748542634581
_C6 = -0.026067544262371012


def _ln(x):
    bits = plsc.bitcast(x, jnp.int32)
    e = (bits >> 23) - 127
    m = plsc.bitcast((bits & 0x7FFFFF) | 0x3F800000, jnp.float32)
    z = m - 1.0
    p = z * (_C1 + z * (_C2 + z * (_C3 + z * (_C4 + z * (_C5 + z * _C6)))))
    return LN2 * (e.astype(jnp.float32) + p)


def _sc_loss_body(sc_hbm, ts_hbm, ob_hbm, tb_hbm, out_hbm,
                  sc_v, ts_v, ob_v, tb_v, ps_v, res_v,
                  sem_s, sem_t, sem_ob, sem_tb):
    wid = lax.axis_index("s") * NC + lax.axis_index("c")
    base_c = wid * CPW

    cp_s = pltpu.async_copy(sc_hbm.at[0, :, pl.ds(wid * 2, 2), :], sc_v, sem_s)
    cp_t = pltpu.async_copy(ts_hbm.at[0, pl.ds(wid * APW, APW)], ts_v, sem_t)
    cp_ob = pltpu.async_copy(ob_hbm.at[0, :, pl.ds(wid * 2, 2), :], ob_v,
                             sem_ob)
    cp_tb = pltpu.async_copy(tb_hbm.at[pl.ds(base_c, CPW)], tb_v, sem_tb)
    scf = sc_v.reshape(18, 64)
    obf = ob_v.reshape(72, 64)

    zeros = jnp.zeros((L,), jnp.float32)
    iota = lax.iota(jnp.int32, L)

    cp_s.wait()
    cp_t.wait()

    @plsc.parallel_loop(0, APW // L, unroll=4,
                        carry=(zeros, zeros, zeros, zeros))
    def score_loop(g, carry):
        accp, accv, accb, accc = carry
        n = g * L + iota
        r = n // 9
        j = n - r * 9
        s = plsc.load_gather(scf, [j * 2 + (r >> 6), r & 63])
        t = ts_v[pl.ds(g * L, L)]
        valid = jnp.where(s != -1.0, 1.0, 0.0)
        ps = jnp.where(s > 0.0, valid, 0.0)
        ps_v[pl.ds(g * L, L)] = ps
        p = jnp.minimum(jnp.maximum(s, EPS), 1.0 - EPS)
        x = jnp.where(t == 1.0, p, 1.0 - p)
        cmask = jnp.where(t != -1.0, 1.0, 0.0)
        return (accp + ps, accv + valid,
                accb + cmask * _ln(x), accc + cmask)

    accp, accv, accb, accc = score_loop

    cp_ob.wait()
    cp_tb.wait()

    @plsc.parallel_loop(0, CPW // L, unroll=4, carry=zeros)
    def box_loop(i, acca):
        off = 128 * (i // 32) + 16 * (i % 8)
        c = (i // 8) % 4
        n = off + iota
        r = n // 9
        j = n - r * 9
        o = plsc.load_gather(obf, [j * 8 + (c * 2 + (r >> 6)), r & 63])
        d = jnp.abs(tb_v[pl.ds(i * L, L)] - o)
        sl1 = jnp.where(d < 1.0, 0.5 * d * d, d - 0.5)
        w = ps_v[pl.ds(off, L)]
        return acca + w * sl1

    acca = box_loop

    res_v[pl.ds(0, L)] = acca
    res_v[pl.ds(L, L)] = accp
    res_v[pl.ds(2 * L, L)] = accv
    res_v[pl.ds(3 * L, L)] = accb
    res_v[pl.ds(4 * L, L)] = accc
    pltpu.sync_copy(res_v, out_hbm.at[wid])


@functools.lru_cache(maxsize=1)
def _sc_loss():
    return pl.kernel(
        _sc_loss_body,
        compiler_params=pltpu.CompilerParams(needs_layout_passes=False,
                                             use_tc_tiling_on_sc=True),
        out_type=jax.ShapeDtypeStruct((NW, 5 * L), jnp.float32),
        mesh=plsc.VectorSubcoreMesh(core_axis_name="c", subcore_axis_name="s",
                                    num_cores=NC, num_subcores=NS),
        scratch_types=[
            pltpu.VMEM((9, 2, 64), jnp.float32),
            pltpu.VMEM((APW,), jnp.float32),
            pltpu.VMEM((36, 2, 64), jnp.float32),
            pltpu.VMEM((CPW,), jnp.float32),
            pltpu.VMEM((APW,), jnp.float32),
            pltpu.VMEM((5 * L,), jnp.float32),
            pltpu.SemaphoreType.DMA,
            pltpu.SemaphoreType.DMA,
            pltpu.SemaphoreType.DMA,
            pltpu.SemaphoreType.DMA,
        ],
    )


def _planar(boxes):
    return boxes.reshape(288, 128, 4).transpose(0, 2, 1).reshape(-1)


def kernel(output_bounding_boxes, target_bounding_boxes, output_scores, target_scores):
    sc_pm = output_scores.transpose(0, 3, 1, 2)
    ob_pm = output_bounding_boxes.transpose(0, 3, 1, 2)
    tb = _planar(target_bounding_boxes)
    ts = target_scores

    partials = _sc_loss()(sc_pm, ts, ob_pm, tb)
    sums = jnp.sum(partials.reshape(NW, 5, L), axis=(0, 2))
    classification_loss = -sums[3] / sums[4]
    regression_loss = 10.0 * (sums[0] / (sums[1] + sums[2] * EPS))
    return classification_loss + regression_loss


# speedup vs baseline: 1.0951x; 1.0951x over previous
import functools

import jax
import jax.numpy as jnp
from jax import lax
from jax.experimental import pallas as pl
from jax.experimental.pallas import tpu as pltpu
from jax.experimental.pallas import tpu_sc as plsc

EPS = 1e-7

N_ANCHORS = 36864
NC, NS, L = 1, 16, 16
NW = NC * NS
APW = N_ANCHORS // NW
CPW = APW * 4


def _sc_regression_body(scores_hbm, ob_hbm, tb_hbm, out_hbm,
                        sc_v, ob_v, tb_v, ps_v, res_v,
                        sem_s, sem_ob, sem_tb):
    wid = lax.axis_index("s")
    base_a = wid * APW
    base_c = wid * CPW

    cp_s = pltpu.async_copy(scores_hbm.at[pl.ds(base_a, APW)], sc_v, sem_s)
    cp_ob = pltpu.async_copy(ob_hbm.at[0, :, pl.ds(wid * 4, 4), :], ob_v,
                             sem_ob)
    cp_tb = pltpu.async_copy(tb_hbm.at[pl.ds(base_c, CPW)], tb_v, sem_tb)
    obf = ob_v.reshape(144, 64)

    zeros = jnp.zeros((L,), jnp.float32)
    lane4 = lax.iota(jnp.int32, L) * 4

    cp_s.wait()

    @plsc.parallel_loop(0, APW // L, unroll=4, carry=(zeros, zeros))
    def score_loop(i, carry):
        accp, accv = carry
        s = sc_v[pl.ds(i * L, L)]
        valid = jnp.where(s != -1.0, 1.0, 0.0)
        ps = jnp.where(s > 0.0, valid, 0.0)
        ps_v[pl.ds(i * L, L)] = ps
        return (accp + ps, accv + valid)

    accp, accv = score_loop

    cp_ob.wait()
    cp_tb.wait()

    iota = lax.iota(jnp.int32, L)

    @plsc.parallel_loop(0, CPW // L, unroll=4, carry=zeros)
    def box_loop(i, acca):
        off = 128 * (i // 32) + 16 * (i % 8)
        c = (i // 8) % 4
        n = off + iota
        r = n // 9
        j = n - r * 9
        o = plsc.load_gather(obf, [j * 16 + (c * 4 + (r >> 6)), r & 63])
        d = jnp.abs(tb_v[pl.ds(i * L, L)] - o)
        sl1 = jnp.where(d < 1.0, 0.5 * d * d, d - 0.5)
        w = ps_v[pl.ds(off, L)]
        return acca + w * sl1

    acca = box_loop

    res_v[pl.ds(0, L)] = acca
    res_v[pl.ds(L, L)] = accp
    res_v[pl.ds(2 * L, L)] = accv
    pltpu.sync_copy(res_v, out_hbm.at[wid])


@functools.lru_cache(maxsize=1)
def _sc_regression():
    return pl.kernel(
        _sc_regression_body,
        compiler_params=pltpu.CompilerParams(needs_layout_passes=False,
                                             use_tc_tiling_on_sc=True),
        out_type=jax.ShapeDtypeStruct((NW, 3 * L), jnp.float32),
        mesh=plsc.VectorSubcoreMesh(core_axis_name="c", subcore_axis_name="s",
                                    num_cores=NC, num_subcores=NS),
        scratch_types=[
            pltpu.VMEM((APW,), jnp.float32),
            pltpu.VMEM((36, 4, 64), jnp.float32),
            pltpu.VMEM((CPW,), jnp.float32),
            pltpu.VMEM((APW,), jnp.float32),
            pltpu.VMEM((3 * L,), jnp.float32),
            pltpu.SemaphoreType.DMA,
            pltpu.SemaphoreType.DMA,
            pltpu.SemaphoreType.DMA,
        ],
    )


def _tc_bce_body(ts_ref, os_ref, part_ref, out_ref):
    t = ts_ref[...]
    p = jnp.clip(os_ref[...], EPS, 1.0 - EPS)
    bce = -(t * jnp.log(p) + (1.0 - t) * jnp.log(1.0 - p))
    mask = (t != -1.0).astype(jnp.float32)
    classification_loss = jnp.sum(bce * mask) / jnp.sum(mask)
    parts = part_ref[...].reshape(NW, 3, L)
    a = jnp.sum(parts[:, 0, :])
    bp = jnp.sum(parts[:, 1, :])
    vm = jnp.sum(parts[:, 2, :])
    regression_loss = 10.0 * (a / (bp + vm * EPS))
    out_ref[0, 0] = classification_loss + regression_loss


def _tc_bce(target_scores_2d, output_scores_2d, partials):
    return pl.pallas_call(
        _tc_bce_body,
        out_shape=jax.ShapeDtypeStruct((1, 1), jnp.float32),
        out_specs=pl.BlockSpec(memory_space=pltpu.SMEM),
    )(target_scores_2d, output_scores_2d, partials)


def _planar(boxes):
    return boxes.reshape(288, 128, 4).transpose(0, 2, 1).reshape(-1)


def kernel(output_bounding_boxes, target_bounding_boxes, output_scores, target_scores):
    scores = output_scores.reshape(-1)
    ob = output_bounding_boxes.transpose(0, 3, 1, 2)
    tb = _planar(target_bounding_boxes)

    partials = _sc_regression()(scores, ob, tb)
    scores_lin = lax.optimization_barrier(scores)
    loss = _tc_bce(target_scores.reshape(288, 128),
                   scores_lin.reshape(288, 128), partials)
    return loss.reshape(())


# speedup vs baseline: 1.1384x; 1.0396x over previous
import functools

import jax
import jax.numpy as jnp
from jax import lax
from jax.experimental import pallas as pl
from jax.experimental.pallas import tpu as pltpu
from jax.experimental.pallas import tpu_sc as plsc

EPS = 1e-7

N_ANCHORS = 36864
NC, NS, L = 2, 16, 16
NW = NC * NS
APW = N_ANCHORS // NW
CPW = APW * 4


def _sc_regression_body(scores_hbm, ob_hbm, tb_hbm, out_hbm,
                        sc_v, ob_v, tb_v, ps_v, res_v,
                        sem_s, sem_ob, sem_tb):
    wid = lax.axis_index("s") * NC + lax.axis_index("c")
    base_a = wid * APW
    base_c = wid * CPW

    cp_s = pltpu.async_copy(scores_hbm.at[pl.ds(base_a, APW)], sc_v, sem_s)
    cp_ob = pltpu.async_copy(ob_hbm.at[0, :, pl.ds(wid * 2, 2), :], ob_v,
                             sem_ob)
    cp_tb = pltpu.async_copy(tb_hbm.at[pl.ds(base_c, CPW)], tb_v, sem_tb)
    obf = ob_v.reshape(72, 64)

    zeros = jnp.zeros((L,), jnp.float32)
    lane4 = lax.iota(jnp.int32, L) * 4

    cp_s.wait()

    @plsc.parallel_loop(0, APW // L, unroll=4, carry=(zeros, zeros))
    def score_loop(i, carry):
        accp, accv = carry
        s = sc_v[pl.ds(i * L, L)]
        valid = jnp.where(s != -1.0, 1.0, 0.0)
        ps = jnp.where(s > 0.0, valid, 0.0)
        ps_v[pl.ds(i * L, L)] = ps
        return (accp + ps, accv + valid)

    accp, accv = score_loop

    cp_ob.wait()
    cp_tb.wait()

    iota = lax.iota(jnp.int32, L)

    @plsc.parallel_loop(0, CPW // L, unroll=8, carry=zeros)
    def box_loop(i, acca):
        off = 128 * (i // 32) + 16 * (i % 8)
        c = (i // 8) % 4
        n = off + iota
        r = n // 9
        j = n - r * 9
        o = plsc.load_gather(obf, [j * 8 + (c * 2 + (r >> 6)), r & 63])
        d = jnp.abs(tb_v[pl.ds(i * L, L)] - o)
        sl1 = jnp.where(d < 1.0, 0.5 * d * d, d - 0.5)
        w = ps_v[pl.ds(off, L)]
        return acca + w * sl1

    acca = box_loop

    res_v[pl.ds(0, L)] = acca
    res_v[pl.ds(L, L)] = accp
    res_v[pl.ds(2 * L, L)] = accv
    pltpu.sync_copy(res_v, out_hbm.at[wid])


@functools.lru_cache(maxsize=1)
def _sc_regression():
    return pl.kernel(
        _sc_regression_body,
        compiler_params=pltpu.CompilerParams(needs_layout_passes=False,
                                             use_tc_tiling_on_sc=True),
        out_type=jax.ShapeDtypeStruct((NW, 3 * L), jnp.float32),
        mesh=plsc.VectorSubcoreMesh(core_axis_name="c", subcore_axis_name="s",
                                    num_cores=NC, num_subcores=NS),
        scratch_types=[
            pltpu.VMEM((APW,), jnp.float32),
            pltpu.VMEM((36, 2, 64), jnp.float32),
            pltpu.VMEM((CPW,), jnp.float32),
            pltpu.VMEM((APW,), jnp.float32),
            pltpu.VMEM((3 * L,), jnp.float32),
            pltpu.SemaphoreType.DMA,
            pltpu.SemaphoreType.DMA,
            pltpu.SemaphoreType.DMA,
        ],
    )


def _tc_bce_body(ts_ref, os_ref, part_ref, out_ref):
    t = ts_ref[...]
    p = jnp.clip(os_ref[...], EPS, 1.0 - EPS)
    bce = -(t * jnp.log(p) + (1.0 - t) * jnp.log(1.0 - p))
    mask = (t != -1.0).astype(jnp.float32)
    classification_loss = jnp.sum(bce * mask) / jnp.sum(mask)
    parts = part_ref[...].reshape(NW, 3, L)
    a = jnp.sum(parts[:, 0, :])
    bp = jnp.sum(parts[:, 1, :])
    vm = jnp.sum(parts[:, 2, :])
    regression_loss = 10.0 * (a / (bp + vm * EPS))
    out_ref[0, 0] = classification_loss + regression_loss


def _tc_bce(target_scores_2d, output_scores_2d, partials):
    return pl.pallas_call(
        _tc_bce_body,
        out_shape=jax.ShapeDtypeStruct((1, 1), jnp.float32),
        out_specs=pl.BlockSpec(memory_space=pltpu.SMEM),
    )(target_scores_2d, output_scores_2d, partials)


def _planar(boxes):
    return boxes.reshape(288, 128, 4).transpose(0, 2, 1).reshape(-1)


def kernel(output_bounding_boxes, target_bounding_boxes, output_scores, target_scores):
    scores = output_scores.reshape(-1)
    ob = output_bounding_boxes.transpose(0, 3, 1, 2)
    tb = _planar(target_bounding_boxes)

    partials = _sc_regression()(scores, ob, tb)
    scores_lin = lax.optimization_barrier(scores)
    loss = _tc_bce(target_scores.reshape(288, 128),
                   scores_lin.reshape(288, 128), partials)
    return loss.reshape(())
